# baseline (speedup 1.0000x reference)
import functools

import jax
import jax.numpy as jnp
from jax import lax
from jax.experimental import pallas as pl
from jax.experimental.pallas import tpu as pltpu
from jax.experimental.pallas import tpu_sc as plsc

NC = 2
NS = 16
L = 16
NW = NC * NS
EDGE_BATCH = 128
PK = 8


@functools.lru_cache(maxsize=None)
def _build_sc_kernel(n_nodes, d_feat, e_pad):
  assert n_nodes % NS == 0 and d_feat % L == 0
  assert e_pad % (NW * EDGE_BATCH * PK) == 0
  epw = e_pad // NW
  n_batches = epw // EDGE_BATCH
  n_pk = n_batches // PK
  zt = 10
  rows_per_tile = n_nodes // zt
  assert n_nodes % zt == 0 and rows_per_tile % 8 == 0
  n_chunks = d_feat // L
  n_groups = EDGE_BATCH // L

  mesh = plsc.VectorSubcoreMesh(core_axis_name="c", subcore_axis_name="s",
                                num_cores=NC, num_subcores=NS)

  @functools.partial(
      pl.kernel,
      out_type=jax.ShapeDtypeStruct((NC, n_nodes, d_feat), jnp.float32),
      mesh=mesh,
      scratch_types=[
          pltpu.VMEM_SHARED((n_nodes, d_feat), jnp.float32),
          pltpu.VMEM((PK * 3 * EDGE_BATCH,), jnp.int32),
          pltpu.VMEM((EDGE_BATCH,), jnp.int32),
          pltpu.VMEM((EDGE_BATCH,), jnp.int32),
          pltpu.VMEM((EDGE_BATCH, d_feat), jnp.float32),
          pltpu.SemaphoreType.DMA,
      ],
  )
  def sc_kernel(x_hbm, pack_hbm, out_hbm, acc, pkv, sidx_s, didx_s, rows_v,
                sem):
    cid = lax.axis_index("c")
    sid = lax.axis_index("s")
    wid = cid * NS + sid

    @pl.when(sid < zt)
    def _():
      def zfill(j, _):
        for c in range(n_chunks):
          rows_v[j, pl.ds(c * L, L)] = jnp.zeros((L,), jnp.float32)
        return 0
      lax.fori_loop(0, EDGE_BATCH, zfill, 0)

      full, rem = divmod(rows_per_tile, EDGE_BATCH)
      for j in range(full):
        pltpu.sync_copy(
            rows_v,
            acc.at[pl.ds(sid * rows_per_tile + j * EDGE_BATCH, EDGE_BATCH)])
      if rem:
        pltpu.sync_copy(
            rows_v.at[pl.ds(0, rem)],
            acc.at[pl.ds(sid * rows_per_tile + full * EDGE_BATCH, rem)])
    plsc.subcore_barrier()

    pkbase = wid * (n_batches * 3 * EDGE_BATCH)

    def pk_chunk(u, _):
      pltpu.sync_copy(
          pack_hbm.at[pl.ds(pkbase + u * (PK * 3 * EDGE_BATCH),
                            PK * 3 * EDGE_BATCH)],
          pkv)
      def pk_batch(j, _):
        base = pl.multiple_of(j * 3 * EDGE_BATCH, EDGE_BATCH)
        for c in range(n_groups):
          sidx_s[pl.ds(c * L, L)] = pkv[pl.ds(base + c * L, L)]
          didx_s[pl.ds(c * L, L)] = pkv[pl.ds(base + EDGE_BATCH + c * L, L)]
        pltpu.async_copy(x_hbm.at[sidx_s], rows_v, sem).wait()

        def scale_group(g, _):
          wvec = pkv[pl.ds(base + 2 * EDGE_BATCH + g * L, L)]
          for jj in range(L):
            s = jnp.full((L,),
                         lax.bitcast_convert_type(wvec[jj], jnp.float32),
                         jnp.float32)
            e = g * L + jj
            for c in range(n_chunks):
              rows_v[e, pl.ds(c * L, L)] = rows_v[e, pl.ds(c * L, L)] * s
          return 0
        lax.fori_loop(0, n_groups, scale_group, 0)

        pltpu.sync_copy(rows_v, acc.at[didx_s], add=True)
        return 0
      lax.fori_loop(0, PK, pk_batch, 0)
      return 0
    lax.fori_loop(0, n_pk, pk_chunk, 0)
    plsc.subcore_barrier()

    @pl.when(sid < zt)
    def _():
      pltpu.sync_copy(
          acc.at[pl.ds(sid * rows_per_tile, rows_per_tile)],
          out_hbm.at[cid, pl.ds(sid * rows_per_tile, rows_per_tile)])

  return sc_kernel


def _combine_body(p_ref, o_ref):
  o_ref[...] = p_ref[0] + p_ref[1]


@functools.lru_cache(maxsize=None)
def _build_combine(n_nodes, d_feat):
  grid = 10 if n_nodes % 80 == 0 else 1
  blk = n_nodes // grid
  return pl.pallas_call(
      _combine_body,
      grid=(grid,),
      in_specs=[pl.BlockSpec((NC, blk, d_feat), lambda i: (0, i, 0))],
      out_specs=pl.BlockSpec((blk, d_feat), lambda i: (i, 0)),
      out_shape=jax.ShapeDtypeStruct((n_nodes, d_feat), jnp.float32),
  )


def kernel(x, edge_index, edge_weight):
  n_nodes, d_feat = x.shape
  n_edges = edge_index.shape[1]
  src = edge_index[0].astype(jnp.int32)
  dst = edge_index[1].astype(jnp.int32)
  w = edge_weight.astype(jnp.float32)

  chunk = NW * EDGE_BATCH * PK
  e_pad = ((n_edges + chunk - 1) // chunk) * chunk
  if e_pad != n_edges:
    pad = e_pad - n_edges
    src = jnp.concatenate([src, jnp.zeros((pad,), jnp.int32)])
    dst = jnp.concatenate([dst, jnp.zeros((pad,), jnp.int32)])
    w = jnp.concatenate([w, jnp.zeros((pad,), jnp.float32)])

  nb_total = e_pad // EDGE_BATCH
  pack = jnp.concatenate(
      [src.reshape(nb_total, EDGE_BATCH),
       dst.reshape(nb_total, EDGE_BATCH),
       lax.bitcast_convert_type(w, jnp.int32).reshape(nb_total, EDGE_BATCH)],
      axis=1).reshape(-1)

  partial = _build_sc_kernel(n_nodes, d_feat, e_pad)(x, pack)
  return _build_combine(n_nodes, d_feat)(partial)


# speedup vs baseline: 1.3947x; 1.3947x over previous
import functools

import jax
import jax.numpy as jnp
from jax import lax
from jax.experimental import pallas as pl
from jax.experimental.pallas import tpu as pltpu
from jax.experimental.pallas import tpu_sc as plsc

NC = 2
NS = 16
L = 16
NW = NC * NS
EDGE_BATCH = 128
PK = 1


@functools.lru_cache(maxsize=None)
def _build_sc_kernel(n_nodes, d_feat, e_pad):
  assert n_nodes % NS == 0 and d_feat % L == 0
  assert e_pad % (NW * EDGE_BATCH * PK) == 0
  epw = e_pad // NW
  n_batches = epw // EDGE_BATCH
  n_pk = n_batches // PK
  zt = 10
  rows_per_tile = n_nodes // zt
  assert n_nodes % zt == 0 and rows_per_tile % 8 == 0
  n_chunks = d_feat // L
  n_groups = EDGE_BATCH // L

  mesh = plsc.VectorSubcoreMesh(core_axis_name="c", subcore_axis_name="s",
                                num_cores=NC, num_subcores=NS)

  @functools.partial(
      pl.kernel,
      out_type=jax.ShapeDtypeStruct((NC, n_nodes, d_feat), jnp.float32),
      mesh=mesh,
      scratch_types=[
          pltpu.VMEM_SHARED((n_nodes, d_feat), jnp.float32),
          pltpu.VMEM((PK * 3 * EDGE_BATCH,), jnp.int32),
          pltpu.VMEM((EDGE_BATCH,), jnp.int32),
          pltpu.VMEM((EDGE_BATCH,), jnp.int32),
          pltpu.VMEM((EDGE_BATCH, d_feat), jnp.float32),
          pltpu.SemaphoreType.DMA,
      ],
  )
  def sc_kernel(x_hbm, pack_hbm, out_hbm, acc, pkv, sidx_s, didx_s, rows_v,
                sem):
    cid = lax.axis_index("c")
    sid = lax.axis_index("s")
    wid = cid * NS + sid

    @pl.when(sid < zt)
    def _():
      def zfill(j, _):
        for c in range(n_chunks):
          rows_v[j, pl.ds(c * L, L)] = jnp.zeros((L,), jnp.float32)
        return 0
      lax.fori_loop(0, EDGE_BATCH, zfill, 0)

      full, rem = divmod(rows_per_tile, EDGE_BATCH)
      for j in range(full):
        pltpu.sync_copy(
            rows_v,
            acc.at[pl.ds(sid * rows_per_tile + j * EDGE_BATCH, EDGE_BATCH)])
      if rem:
        pltpu.sync_copy(
            rows_v.at[pl.ds(0, rem)],
            acc.at[pl.ds(sid * rows_per_tile + full * EDGE_BATCH, rem)])
    plsc.subcore_barrier()

    pkbase = wid * (n_batches * 3 * EDGE_BATCH)

    def pk_chunk(u, _):
      pltpu.sync_copy(
          pack_hbm.at[pl.ds(pkbase + u * (PK * 3 * EDGE_BATCH),
                            PK * 3 * EDGE_BATCH)],
          pkv)
      def pk_batch(j, _):
        base = pl.multiple_of(j * 3 * EDGE_BATCH, EDGE_BATCH)
        for c in range(n_groups):
          sidx_s[pl.ds(c * L, L)] = pkv[pl.ds(base + c * L, L)]
          didx_s[pl.ds(c * L, L)] = pkv[pl.ds(base + EDGE_BATCH + c * L, L)]
        pltpu.async_copy(x_hbm.at[sidx_s], rows_v, sem).wait()

        def scale_group(g, _):
          wvec = pkv[pl.ds(base + 2 * EDGE_BATCH + g * L, L)]
          for jj in range(L):
            s = jnp.full((L,),
                         lax.bitcast_convert_type(wvec[jj], jnp.float32),
                         jnp.float32)
            e = g * L + jj
            for c in range(n_chunks):
              rows_v[e, pl.ds(c * L, L)] = rows_v[e, pl.ds(c * L, L)] * s
          return 0
        lax.fori_loop(0, n_groups, scale_group, 0)

        pltpu.sync_copy(rows_v, acc.at[didx_s], add=True)
        return 0
      lax.fori_loop(0, PK, pk_batch, 0)
      return 0
    lax.fori_loop(0, n_pk, pk_chunk, 0)
    plsc.subcore_barrier()

    @pl.when(sid < zt)
    def _():
      pltpu.sync_copy(
          acc.at[pl.ds(sid * rows_per_tile, rows_per_tile)],
          out_hbm.at[cid, pl.ds(sid * rows_per_tile, rows_per_tile)])

  return sc_kernel


def _combine_body(p_ref, o_ref):
  o_ref[...] = p_ref[0] + p_ref[1]


@functools.lru_cache(maxsize=None)
def _build_combine(n_nodes, d_feat):
  grid = 10 if n_nodes % 80 == 0 else 1
  blk = n_nodes // grid
  return pl.pallas_call(
      _combine_body,
      grid=(grid,),
      in_specs=[pl.BlockSpec((NC, blk, d_feat), lambda i: (0, i, 0))],
      out_specs=pl.BlockSpec((blk, d_feat), lambda i: (i, 0)),
      out_shape=jax.ShapeDtypeStruct((n_nodes, d_feat), jnp.float32),
  )


def kernel(x, edge_index, edge_weight):
  n_nodes, d_feat = x.shape
  n_edges = edge_index.shape[1]
  src = edge_index[0].astype(jnp.int32)
  dst = edge_index[1].astype(jnp.int32)
  w = edge_weight.astype(jnp.float32)

  chunk = NW * EDGE_BATCH * PK
  e_pad = ((n_edges + chunk - 1) // chunk) * chunk
  if e_pad != n_edges:
    pad = e_pad - n_edges
    src = jnp.concatenate([src, jnp.zeros((pad,), jnp.int32)])
    dst = jnp.concatenate([dst, jnp.zeros((pad,), jnp.int32)])
    w = jnp.concatenate([w, jnp.zeros((pad,), jnp.float32)])

  nb_total = e_pad // EDGE_BATCH
  pack = jnp.concatenate(
      [src.reshape(nb_total, EDGE_BATCH),
       dst.reshape(nb_total, EDGE_BATCH),
       lax.bitcast_convert_type(w, jnp.int32).reshape(nb_total, EDGE_BATCH)],
      axis=1).reshape(-1)

  partial = _build_sc_kernel(n_nodes, d_feat, e_pad)(x, pack)
  return _build_combine(n_nodes, d_feat)(partial)
